# baseline (speedup 1.0000x reference)
import functools

import jax
import jax.numpy as jnp
from jax import lax
from jax.experimental import pallas as pl
from jax.experimental.pallas import tpu as pltpu
from jax.experimental.pallas import tpu_sc as plsc

N = 10000
E = 320000
D = 128

NC = 2
NS = 16
NW = NC * NS
C = 128
NCHUNK = 78
NPAIR = NCHUNK // 2
XTRA_WORKERS = E // C - NW * NCHUNK
WB = 80
RPT = 640
RPT_LAST_CHUNKS = (N - (NS - 1) * RPT) // WB
RPT_CHUNKS = RPT // WB


def _mm_body(f_ref, w_ref, b_ref, o_ref):
    o_ref[...] = (
        jax.lax.dot_general(
            f_ref[...], w_ref[...], (((1,), (1,)), ((), ())),
            preferred_element_type=jnp.float32)
        + b_ref[...]
    )


def _add_body(p_ref, o_ref):
    o_ref[...] = p_ref[0] + p_ref[1]


def _bcast_lane(v, lane):
    return lax.gather(
        v, jnp.full((16, 1), lane, jnp.int32),
        lax.GatherDimensionNumbers(
            offset_dims=(), collapsed_slice_dims=(0,), start_index_map=(0,)),
        (1,), mode=lax.GatherScatterMode.PROMISE_IN_BOUNDS)


def _sc_spmm_body(support_hbm, lap_hbm, vals_hbm, out_hbm,
                  lbufa, lbufb, lbufm, vbufa, vbufb, bufa, bufb, acc,
                  gsema, gsemb, esema, esemb, ssem):
    cid = lax.axis_index("c")
    sid = lax.axis_index("s")
    wid = cid * NS + sid
    base_w = (NCHUNK * wid + jnp.minimum(wid, XTRA_WORKERS)) * C
    zero16 = jnp.zeros((16,), jnp.float32)

    def zrow(g, carry):
        for j in range(D // 16):
            bufa[g, pl.ds(j * 16, 16)] = zero16
        return carry
    lax.fori_loop(0, C, zrow, 0)

    row0 = sid * RPT
    nwb = jnp.where(sid == NS - 1, RPT_LAST_CHUNKS, RPT_CHUNKS)

    def zacc(k, carry):
        pltpu.sync_copy(bufa.at[pl.ds(0, WB)], acc.at[pl.ds(row0 + k * WB, WB)])
        return carry
    lax.fori_loop(0, nwb, zacc, 0)

    plsc.subcore_barrier()

    def start_e(ci, lbuf, vbuf, esem):
        pltpu.async_copy(lap_hbm.at[:, pl.ds(base_w + ci * C, C)], lbuf, esem)
        pltpu.async_copy(vals_hbm.at[pl.ds(base_w + ci * C, C)], vbuf, esem)

    def wait_e(ci, lbuf, vbuf, esem):
        pltpu.make_async_copy(
            lap_hbm.at[:, pl.ds(base_w + ci * C, C)], lbuf, esem).wait()
        pltpu.make_async_copy(
            vals_hbm.at[pl.ds(base_w + ci * C, C)], vbuf, esem).wait()

    def start_g(buf, lbuf, gsem):
        pltpu.async_copy(support_hbm.at[lbuf.at[1]], buf, gsem)

    def wait_g(buf, lbuf, gsem):
        pltpu.make_async_copy(support_hbm.at[lbuf.at[1]], buf, gsem).wait()

    def scale(buf, vbuf, ngroup):
        def body(g, carry):
            vv = vbuf[pl.ds(g * 16, 16)]
            for i2 in range(16):
                r = g * 16 + i2
                s = _bcast_lane(vv, i2)
                for j in range(D // 16):
                    buf[r, pl.ds(j * 16, 16)] = buf[r, pl.ds(j * 16, 16)] * s
            return carry
        lax.fori_loop(0, ngroup, body, 0)

    def scatter(buf, rbuf):
        pltpu.async_copy(buf, acc.at[rbuf], ssem, add=True).wait()

    start_e(0, lbufa, vbufa, esema)
    start_e(1, lbufb, vbufb, esemb)
    wait_e(0, lbufa, vbufa, esema)
    start_g(bufa, lbufa, gsema)

    def pair(k, carry):
        ci = 2 * k
        wait_g(bufa, lbufa, gsema)
        wait_e(ci + 1, lbufb, vbufb, esemb)
        start_g(bufb, lbufb, gsemb)
        scale(bufa, vbufa, C // 16)
        scatter(bufa, lbufa.at[0])

        @pl.when(k < NPAIR - 1)
        def _():
            start_e(ci + 2, lbufa, vbufa, esema)
        wait_g(bufb, lbufb, gsemb)

        @pl.when(k < NPAIR - 1)
        def _():
            wait_e(ci + 2, lbufa, vbufa, esema)
            start_g(bufa, lbufa, gsema)
        scale(bufb, vbufb, C // 16)
        scatter(bufb, lbufb.at[0])

        @pl.when(k < NPAIR - 1)
        def _():
            start_e(ci + 3, lbufb, vbufb, esemb)
        return carry
    lax.fori_loop(0, NPAIR, pair, 0)

    @pl.when(wid < XTRA_WORKERS)
    def _():
        pltpu.sync_copy(lap_hbm.at[:, pl.ds(base_w + NCHUNK * C, C)], lbufm)
        pltpu.sync_copy(vals_hbm.at[pl.ds(base_w + NCHUNK * C, C)], vbufa)
        pltpu.async_copy(support_hbm.at[lbufm.at[1]], bufa, gsema).wait()
        scale(bufa, vbufa, C // 16)
        pltpu.async_copy(bufa, acc.at[lbufm.at[0]], ssem, add=True).wait()

    plsc.subcore_barrier()

    def wb(k, carry):
        sl = pl.ds(row0 + k * WB, WB)
        pltpu.sync_copy(acc.at[sl], bufa.at[pl.ds(0, WB)])
        pltpu.sync_copy(bufa.at[pl.ds(0, WB)], out_hbm.at[cid, sl])
        return carry
    lax.fori_loop(0, nwb, wb, 0)


_sc_spmm = functools.partial(
    pl.kernel,
    out_type=jax.ShapeDtypeStruct((NC, N, D), jnp.float32),
    mesh=plsc.VectorSubcoreMesh(
        core_axis_name="c", subcore_axis_name="s",
        num_cores=NC, num_subcores=NS),
    scratch_types=[
        pltpu.VMEM((2, C), jnp.int32),
        pltpu.VMEM((2, C), jnp.int32),
        pltpu.VMEM((2, C), jnp.int32),
        pltpu.VMEM((C,), jnp.float32),
        pltpu.VMEM((C,), jnp.float32),
        pltpu.VMEM((C, D), jnp.float32),
        pltpu.VMEM((C, D), jnp.float32),
        pltpu.VMEM_SHARED((N, D), jnp.float32),
        pltpu.SemaphoreType.DMA,
        pltpu.SemaphoreType.DMA,
        pltpu.SemaphoreType.DMA,
        pltpu.SemaphoreType.DMA,
        pltpu.SemaphoreType.DMA,
    ],
)(_sc_spmm_body)


def kernel(laplacian_indices, laplacian_values, features, W, b):
    b2 = b.reshape(1, D)

    support = pl.pallas_call(
        _mm_body,
        grid=(5,),
        in_specs=[
            pl.BlockSpec((N // 5, D), lambda i: (i, 0)),
            pl.BlockSpec((D, D), lambda i: (0, 0)),
            pl.BlockSpec((1, D), lambda i: (0, 0)),
        ],
        out_specs=pl.BlockSpec((N // 5, D), lambda i: (i, 0)),
        out_shape=jax.ShapeDtypeStruct((N, D), jnp.float32),
    )(features, W, b2)

    partials = _sc_spmm(support, laplacian_indices, laplacian_values)

    out = pl.pallas_call(
        _add_body,
        grid=(10,),
        in_specs=[pl.BlockSpec((NC, N // 10, D), lambda i: (0, i, 0))],
        out_specs=pl.BlockSpec((N // 10, D), lambda i: (i, 0)),
        out_shape=jax.ShapeDtypeStruct((N, D), jnp.float32),
    )(partials)
    return out


# speedup vs baseline: 1.0637x; 1.0637x over previous
import functools

import jax
import jax.numpy as jnp
from jax import lax
from jax.experimental import pallas as pl
from jax.experimental.pallas import tpu as pltpu
from jax.experimental.pallas import tpu_sc as plsc

N = 10000
E = 320000
D = 128

NC = 2
NS = 16
NW = NC * NS
C = 128
NCHUNK = 78
NPAIR = NCHUNK // 2
XTRA_WORKERS = E // C - NW * NCHUNK
WB = 80
RPT = 640
RPT_LAST_CHUNKS = (N - (NS - 1) * RPT) // WB
RPT_CHUNKS = RPT // WB


def _mm_body(f_ref, w_ref, b_ref, o_ref):
    o_ref[...] = (
        jax.lax.dot_general(
            f_ref[...], w_ref[...], (((1,), (1,)), ((), ())),
            preferred_element_type=jnp.float32)
        + b_ref[...]
    )


def _add_body(p_ref, o_ref):
    o_ref[...] = p_ref[0] + p_ref[1]


def _bcast_lane(v, lane):
    return lax.gather(
        v, jnp.full((16, 1), lane, jnp.int32),
        lax.GatherDimensionNumbers(
            offset_dims=(), collapsed_slice_dims=(0,), start_index_map=(0,)),
        (1,), mode=lax.GatherScatterMode.PROMISE_IN_BOUNDS)


def _sc_spmm_body(support_hbm, lap_hbm, cols_hbm, vals_hbm, out_hbm,
                  cbuf, rbufa, rbufb, lbufm, vbufa, vbufb, bufa, bufb, acc,
                  gsema, gsemb, esema, esemb, ssem):
    cid = lax.axis_index("c")
    sid = lax.axis_index("s")
    wid = cid * NS + sid
    base_w = (NCHUNK * wid + jnp.minimum(wid, XTRA_WORKERS)) * C
    zero16 = jnp.zeros((16,), jnp.float32)

    pltpu.sync_copy(cols_hbm.at[pl.ds(base_w, NCHUNK * C)], cbuf)

    def zrow(g, carry):
        for j in range(D // 16):
            bufa[g, pl.ds(j * 16, 16)] = zero16
        return carry
    lax.fori_loop(0, C, zrow, 0)

    row0 = sid * RPT
    nwb = jnp.where(sid == NS - 1, RPT_LAST_CHUNKS, RPT_CHUNKS)

    def zacc(k, carry):
        pltpu.sync_copy(bufa.at[pl.ds(0, WB)], acc.at[pl.ds(row0 + k * WB, WB)])
        return carry
    lax.fori_loop(0, nwb, zacc, 0)

    plsc.subcore_barrier()

    def start_all(ci, buf, rbuf, vbuf, gsem, esem):
        pltpu.async_copy(
            support_hbm.at[cbuf.at[pl.ds(ci * C, C)]], buf, gsem)
        pltpu.async_copy(lap_hbm.at[0, pl.ds(base_w + ci * C, C)], rbuf, esem)
        pltpu.async_copy(vals_hbm.at[pl.ds(base_w + ci * C, C)], vbuf, esem)

    def wait_all(ci, buf, rbuf, vbuf, gsem, esem):
        pltpu.make_async_copy(
            support_hbm.at[cbuf.at[pl.ds(ci * C, C)]], buf, gsem).wait()
        pltpu.make_async_copy(
            lap_hbm.at[0, pl.ds(base_w + ci * C, C)], rbuf, esem).wait()
        pltpu.make_async_copy(
            vals_hbm.at[pl.ds(base_w + ci * C, C)], vbuf, esem).wait()

    def scale(buf, vbuf, ngroup):
        def body(g, carry):
            vv = vbuf[pl.ds(g * 16, 16)]
            for i2 in range(16):
                r = g * 16 + i2
                s = _bcast_lane(vv, i2)
                for j in range(D // 16):
                    buf[r, pl.ds(j * 16, 16)] = buf[r, pl.ds(j * 16, 16)] * s
            return carry
        lax.fori_loop(0, ngroup, body, 0)

    def scatter(buf, rbuf):
        pltpu.async_copy(buf, acc.at[rbuf], ssem, add=True).wait()

    start_all(0, bufa, rbufa, vbufa, gsema, esema)

    def pair(k, carry):
        ci = 2 * k
        wait_all(ci, bufa, rbufa, vbufa, gsema, esema)
        start_all(ci + 1, bufb, rbufb, vbufb, gsemb, esemb)
        scale(bufa, vbufa, C // 16)
        scatter(bufa, rbufa)
        wait_all(ci + 1, bufb, rbufb, vbufb, gsemb, esemb)

        @pl.when(k < NPAIR - 1)
        def _():
            start_all(ci + 2, bufa, rbufa, vbufa, gsema, esema)
        scale(bufb, vbufb, C // 16)
        scatter(bufb, rbufb)
        return carry
    lax.fori_loop(0, NPAIR, pair, 0)

    @pl.when(wid < XTRA_WORKERS)
    def _():
        pltpu.sync_copy(lap_hbm.at[:, pl.ds(base_w + NCHUNK * C, C)], lbufm)
        pltpu.sync_copy(vals_hbm.at[pl.ds(base_w + NCHUNK * C, C)], vbufa)
        pltpu.async_copy(support_hbm.at[lbufm.at[1]], bufa, gsema).wait()
        scale(bufa, vbufa, C // 16)
        pltpu.async_copy(bufa, acc.at[lbufm.at[0]], ssem, add=True).wait()

    plsc.subcore_barrier()

    def wb(k, carry):
        sl = pl.ds(row0 + k * WB, WB)
        pltpu.sync_copy(acc.at[sl], bufa.at[pl.ds(0, WB)])
        pltpu.sync_copy(bufa.at[pl.ds(0, WB)], out_hbm.at[cid, sl])
        return carry
    lax.fori_loop(0, nwb, wb, 0)


_sc_spmm = functools.partial(
    pl.kernel,
    out_type=jax.ShapeDtypeStruct((NC, N, D), jnp.float32),
    mesh=plsc.VectorSubcoreMesh(
        core_axis_name="c", subcore_axis_name="s",
        num_cores=NC, num_subcores=NS),
    scratch_types=[
        pltpu.VMEM((NCHUNK * C,), jnp.int32),
        pltpu.VMEM((C,), jnp.int32),
        pltpu.VMEM((C,), jnp.int32),
        pltpu.VMEM((2, C), jnp.int32),
        pltpu.VMEM((C,), jnp.float32),
        pltpu.VMEM((C,), jnp.float32),
        pltpu.VMEM((C, D), jnp.float32),
        pltpu.VMEM((C, D), jnp.float32),
        pltpu.VMEM_SHARED((N, D), jnp.float32),
        pltpu.SemaphoreType.DMA,
        pltpu.SemaphoreType.DMA,
        pltpu.SemaphoreType.DMA,
        pltpu.SemaphoreType.DMA,
        pltpu.SemaphoreType.DMA,
    ],
)(_sc_spmm_body)


def kernel(laplacian_indices, laplacian_values, features, W, b):
    b2 = b.reshape(1, D)

    support = pl.pallas_call(
        _mm_body,
        grid=(5,),
        in_specs=[
            pl.BlockSpec((N // 5, D), lambda i: (i, 0)),
            pl.BlockSpec((D, D), lambda i: (0, 0)),
            pl.BlockSpec((1, D), lambda i: (0, 0)),
        ],
        out_specs=pl.BlockSpec((N // 5, D), lambda i: (i, 0)),
        out_shape=jax.ShapeDtypeStruct((N, D), jnp.float32),
    )(features, W, b2)

    partials = _sc_spmm(support, laplacian_indices, laplacian_indices[1],
                        laplacian_values)

    out = pl.pallas_call(
        _add_body,
        grid=(10,),
        in_specs=[pl.BlockSpec((NC, N // 10, D), lambda i: (0, i, 0))],
        out_specs=pl.BlockSpec((N // 10, D), lambda i: (i, 0)),
        out_shape=jax.ShapeDtypeStruct((N, D), jnp.float32),
    )(partials)
    return out


# speedup vs baseline: 1.1526x; 1.0836x over previous
import functools

import jax
import jax.numpy as jnp
from jax import lax
from jax.experimental import pallas as pl
from jax.experimental.pallas import tpu as pltpu
from jax.experimental.pallas import tpu_sc as plsc

N = 10000
E = 320000
D = 128

NC = 2
NS = 16
NW = NC * NS
C = 128
NCHUNK = 78
NPAIR = NCHUNK // 2
XTRA_WORKERS = E // C - NW * NCHUNK
WB = 80
RPT = 640
RPT_LAST_CHUNKS = (N - (NS - 1) * RPT) // WB
RPT_CHUNKS = RPT // WB


def _mm_body(f_ref, w_ref, b_ref, lap_ref, o_ref, cols_ref):
    o_ref[...] = (
        jax.lax.dot_general(
            f_ref[...], w_ref[...], (((1,), (1,)), ((), ())),
            preferred_element_type=jnp.float32)
        + b_ref[...]
    )
    @pl.when(pl.program_id(0) == 0)
    def _():
        cols_ref[...] = lap_ref[1, :]


def _add_body(p_ref, o_ref):
    o_ref[...] = p_ref[0] + p_ref[1]


def _bcast_lane(v, lane):
    return lax.gather(
        v, jnp.full((16, 1), lane, jnp.int32),
        lax.GatherDimensionNumbers(
            offset_dims=(), collapsed_slice_dims=(0,), start_index_map=(0,)),
        (1,), mode=lax.GatherScatterMode.PROMISE_IN_BOUNDS)


def _sc_spmm_body(support_hbm, lap_hbm, cols_hbm, vals_hbm, out_hbm,
                  cbuf, rbufa, rbufb, lbufm, vbufa, vbufb, bufa, bufb, acc,
                  gsema, gsemb, esema, esemb, ssem):
    cid = lax.axis_index("c")
    sid = lax.axis_index("s")
    wid = cid * NS + sid
    base_w = (NCHUNK * wid + jnp.minimum(wid, XTRA_WORKERS)) * C
    zero16 = jnp.zeros((16,), jnp.float32)

    pltpu.sync_copy(cols_hbm.at[pl.ds(base_w, NCHUNK * C)], cbuf)

    def zrow(g, carry):
        for j in range(D // 16):
            bufa[g, pl.ds(j * 16, 16)] = zero16
        return carry
    lax.fori_loop(0, C, zrow, 0)

    row0 = sid * RPT
    nwb = jnp.where(sid == NS - 1, RPT_LAST_CHUNKS, RPT_CHUNKS)

    def zacc(k, carry):
        pltpu.sync_copy(bufa.at[pl.ds(0, WB)], acc.at[pl.ds(row0 + k * WB, WB)])
        return carry
    lax.fori_loop(0, nwb, zacc, 0)

    plsc.subcore_barrier()

    def start_all(ci, buf, rbuf, vbuf, gsem, esem):
        pltpu.async_copy(
            support_hbm.at[cbuf.at[pl.ds(ci * C, C)]], buf, gsem)
        pltpu.async_copy(lap_hbm.at[0, pl.ds(base_w + ci * C, C)], rbuf, esem)
        pltpu.async_copy(vals_hbm.at[pl.ds(base_w + ci * C, C)], vbuf, esem)

    def wait_all(ci, buf, rbuf, vbuf, gsem, esem):
        pltpu.make_async_copy(
            support_hbm.at[cbuf.at[pl.ds(ci * C, C)]], buf, gsem).wait()
        pltpu.make_async_copy(
            lap_hbm.at[0, pl.ds(base_w + ci * C, C)], rbuf, esem).wait()
        pltpu.make_async_copy(
            vals_hbm.at[pl.ds(base_w + ci * C, C)], vbuf, esem).wait()

    def scale(buf, vbuf, ngroup):
        def body(g, carry):
            vv = vbuf[pl.ds(g * 16, 16)]
            for i2 in range(16):
                r = g * 16 + i2
                s = _bcast_lane(vv, i2)
                for j in range(D // 16):
                    buf[r, pl.ds(j * 16, 16)] = buf[r, pl.ds(j * 16, 16)] * s
            return carry
        lax.fori_loop(0, ngroup, body, 0)

    def scatter(buf, rbuf):
        pltpu.async_copy(buf, acc.at[rbuf], ssem, add=True).wait()

    start_all(0, bufa, rbufa, vbufa, gsema, esema)

    def pair(k, carry):
        ci = 2 * k
        wait_all(ci, bufa, rbufa, vbufa, gsema, esema)
        start_all(ci + 1, bufb, rbufb, vbufb, gsemb, esemb)
        scale(bufa, vbufa, C // 16)
        scatter(bufa, rbufa)
        wait_all(ci + 1, bufb, rbufb, vbufb, gsemb, esemb)

        @pl.when(k < NPAIR - 1)
        def _():
            start_all(ci + 2, bufa, rbufa, vbufa, gsema, esema)
        scale(bufb, vbufb, C // 16)
        scatter(bufb, rbufb)
        return carry
    lax.fori_loop(0, NPAIR, pair, 0)

    @pl.when(wid < XTRA_WORKERS)
    def _():
        pltpu.sync_copy(lap_hbm.at[:, pl.ds(base_w + NCHUNK * C, C)], lbufm)
        pltpu.sync_copy(vals_hbm.at[pl.ds(base_w + NCHUNK * C, C)], vbufa)
        pltpu.async_copy(support_hbm.at[lbufm.at[1]], bufa, gsema).wait()
        scale(bufa, vbufa, C // 16)
        pltpu.async_copy(bufa, acc.at[lbufm.at[0]], ssem, add=True).wait()

    plsc.subcore_barrier()

    def wb(k, carry):
        sl = pl.ds(row0 + k * WB, WB)
        pltpu.sync_copy(acc.at[sl], bufa.at[pl.ds(0, WB)])
        pltpu.sync_copy(bufa.at[pl.ds(0, WB)], out_hbm.at[cid, sl])
        return carry
    lax.fori_loop(0, nwb, wb, 0)


_sc_spmm = functools.partial(
    pl.kernel,
    out_type=jax.ShapeDtypeStruct((NC, N, D), jnp.float32),
    mesh=plsc.VectorSubcoreMesh(
        core_axis_name="c", subcore_axis_name="s",
        num_cores=NC, num_subcores=NS),
    scratch_types=[
        pltpu.VMEM((NCHUNK * C,), jnp.int32),
        pltpu.VMEM((C,), jnp.int32),
        pltpu.VMEM((C,), jnp.int32),
        pltpu.VMEM((2, C), jnp.int32),
        pltpu.VMEM((C,), jnp.float32),
        pltpu.VMEM((C,), jnp.float32),
        pltpu.VMEM((C, D), jnp.float32),
        pltpu.VMEM((C, D), jnp.float32),
        pltpu.VMEM_SHARED((N, D), jnp.float32),
        pltpu.SemaphoreType.DMA,
        pltpu.SemaphoreType.DMA,
        pltpu.SemaphoreType.DMA,
        pltpu.SemaphoreType.DMA,
        pltpu.SemaphoreType.DMA,
    ],
)(_sc_spmm_body)


def kernel(laplacian_indices, laplacian_values, features, W, b):
    b2 = b.reshape(1, D)

    support, cols_flat = pl.pallas_call(
        _mm_body,
        grid=(5,),
        in_specs=[
            pl.BlockSpec((N // 5, D), lambda i: (i, 0)),
            pl.BlockSpec((D, D), lambda i: (0, 0)),
            pl.BlockSpec((1, D), lambda i: (0, 0)),
            pl.BlockSpec((2, E), lambda i: (0, 0)),
        ],
        out_specs=[
            pl.BlockSpec((N // 5, D), lambda i: (i, 0)),
            pl.BlockSpec((E,), lambda i: (0,)),
        ],
        out_shape=[
            jax.ShapeDtypeStruct((N, D), jnp.float32),
            jax.ShapeDtypeStruct((E,), jnp.int32),
        ],
    )(features, W, b2, laplacian_indices)

    partials = _sc_spmm(support, laplacian_indices, cols_flat,
                        laplacian_values)

    out = pl.pallas_call(
        _add_body,
        grid=(10,),
        in_specs=[pl.BlockSpec((NC, N // 10, D), lambda i: (0, i, 0))],
        out_specs=pl.BlockSpec((N // 10, D), lambda i: (i, 0)),
        out_shape=jax.ShapeDtypeStruct((N, D), jnp.float32),
    )(partials)
    return out


# speedup vs baseline: 1.1548x; 1.0019x over previous
import functools

import jax
import jax.numpy as jnp
from jax import lax
from jax.experimental import pallas as pl
from jax.experimental.pallas import tpu as pltpu
from jax.experimental.pallas import tpu_sc as plsc

N = 10000
E = 320000
D = 128

NC = 2
NS = 16
NW = NC * NS
C = 128
NCHUNK = 78
NPAIR = NCHUNK // 2
XTRA_WORKERS = E // C - NW * NCHUNK
WB = 80
RPT = 640
RPT_LAST_CHUNKS = (N - (NS - 1) * RPT) // WB
RPT_CHUNKS = RPT // WB


def _mm_body(f_ref, w_ref, b_ref, lap_ref, o_ref, cols_ref):
    o_ref[...] = (
        jax.lax.dot_general(
            f_ref[...], w_ref[...], (((1,), (1,)), ((), ())),
            preferred_element_type=jnp.float32)
        + b_ref[...]
    )
    @pl.when(pl.program_id(0) == 0)
    def _():
        cols_ref[...] = lap_ref[1, :]


def _add_body(p_ref, o_ref):
    o_ref[...] = p_ref[0] + p_ref[1]


def _bcast_lane(v, lane):
    return lax.gather(
        v, jnp.full((16, 1), lane, jnp.int32),
        lax.GatherDimensionNumbers(
            offset_dims=(), collapsed_slice_dims=(0,), start_index_map=(0,)),
        (1,), mode=lax.GatherScatterMode.PROMISE_IN_BOUNDS)


def _sc_spmm_body(support_hbm, lap_hbm, cols_hbm, vals_hbm, out_hbm,
                  cbuf, rbufa, rbufb, lbufm, vbufa, vbufb, bufa, bufb, acc,
                  gsema, gsemb, esema, esemb, ssem):
    cid = lax.axis_index("c")
    sid = lax.axis_index("s")
    wid = cid * NS + sid
    base_w = (NCHUNK * wid + jnp.minimum(wid, XTRA_WORKERS)) * C
    zero16 = jnp.zeros((16,), jnp.float32)

    pltpu.async_copy(cols_hbm.at[pl.ds(base_w, NCHUNK * C)], cbuf, esema)

    def zrow(g, carry):
        for j in range(D // 16):
            bufa[g, pl.ds(j * 16, 16)] = zero16
        return carry
    lax.fori_loop(0, C, zrow, 0)

    row0 = sid * RPT
    nwb = jnp.where(sid == NS - 1, RPT_LAST_CHUNKS, RPT_CHUNKS)

    def zacc(k, carry):
        pltpu.sync_copy(bufa.at[pl.ds(0, WB)], acc.at[pl.ds(row0 + k * WB, WB)])
        return carry
    lax.fori_loop(0, nwb, zacc, 0)

    plsc.subcore_barrier()
    pltpu.make_async_copy(
        cols_hbm.at[pl.ds(base_w, NCHUNK * C)], cbuf, esema).wait()

    def start_all(ci, buf, rbuf, vbuf, gsem, esem):
        pltpu.async_copy(
            support_hbm.at[cbuf.at[pl.ds(ci * C, C)]], buf, gsem)
        pltpu.async_copy(lap_hbm.at[0, pl.ds(base_w + ci * C, C)], rbuf, esem)
        pltpu.async_copy(vals_hbm.at[pl.ds(base_w + ci * C, C)], vbuf, esem)

    def wait_all(ci, buf, rbuf, vbuf, gsem, esem):
        pltpu.make_async_copy(
            support_hbm.at[cbuf.at[pl.ds(ci * C, C)]], buf, gsem).wait()
        pltpu.make_async_copy(
            lap_hbm.at[0, pl.ds(base_w + ci * C, C)], rbuf, esem).wait()
        pltpu.make_async_copy(
            vals_hbm.at[pl.ds(base_w + ci * C, C)], vbuf, esem).wait()

    def scale(buf, vbuf, ngroup):
        def body(g, carry):
            vv = vbuf[pl.ds(g * 16, 16)]
            for i2 in range(16):
                r = g * 16 + i2
                s = _bcast_lane(vv, i2)
                for j in range(D // 16):
                    buf[r, pl.ds(j * 16, 16)] = buf[r, pl.ds(j * 16, 16)] * s
            return carry
        lax.fori_loop(0, ngroup, body, 0)

    def scatter(buf, rbuf):
        pltpu.async_copy(buf, acc.at[rbuf], ssem, add=True).wait()

    start_all(0, bufa, rbufa, vbufa, gsema, esema)

    def pair(k, carry):
        ci = 2 * k
        wait_all(ci, bufa, rbufa, vbufa, gsema, esema)
        start_all(ci + 1, bufb, rbufb, vbufb, gsemb, esemb)
        scale(bufa, vbufa, C // 16)
        scatter(bufa, rbufa)
        wait_all(ci + 1, bufb, rbufb, vbufb, gsemb, esemb)

        @pl.when(k < NPAIR - 1)
        def _():
            start_all(ci + 2, bufa, rbufa, vbufa, gsema, esema)
        scale(bufb, vbufb, C // 16)
        scatter(bufb, rbufb)
        return carry
    lax.fori_loop(0, NPAIR, pair, 0)

    @pl.when(wid < XTRA_WORKERS)
    def _():
        pltpu.sync_copy(lap_hbm.at[:, pl.ds(base_w + NCHUNK * C, C)], lbufm)
        pltpu.sync_copy(vals_hbm.at[pl.ds(base_w + NCHUNK * C, C)], vbufa)
        pltpu.async_copy(support_hbm.at[lbufm.at[1]], bufa, gsema).wait()
        scale(bufa, vbufa, C // 16)
        pltpu.async_copy(bufa, acc.at[lbufm.at[0]], ssem, add=True).wait()

    plsc.subcore_barrier()

    def wb(k, carry):
        sl = pl.ds(row0 + k * WB, WB)
        pltpu.sync_copy(acc.at[sl], out_hbm.at[cid, sl])
        return carry
    lax.fori_loop(0, nwb, wb, 0)


_sc_spmm = functools.partial(
    pl.kernel,
    out_type=jax.ShapeDtypeStruct((NC, N, D), jnp.float32),
    mesh=plsc.VectorSubcoreMesh(
        core_axis_name="c", subcore_axis_name="s",
        num_cores=NC, num_subcores=NS),
    scratch_types=[
        pltpu.VMEM((NCHUNK * C,), jnp.int32),
        pltpu.VMEM((C,), jnp.int32),
        pltpu.VMEM((C,), jnp.int32),
        pltpu.VMEM((2, C), jnp.int32),
        pltpu.VMEM((C,), jnp.float32),
        pltpu.VMEM((C,), jnp.float32),
        pltpu.VMEM((C, D), jnp.float32),
        pltpu.VMEM((C, D), jnp.float32),
        pltpu.VMEM_SHARED((N, D), jnp.float32),
        pltpu.SemaphoreType.DMA,
        pltpu.SemaphoreType.DMA,
        pltpu.SemaphoreType.DMA,
        pltpu.SemaphoreType.DMA,
        pltpu.SemaphoreType.DMA,
    ],
)(_sc_spmm_body)


def kernel(laplacian_indices, laplacian_values, features, W, b):
    b2 = b.reshape(1, D)

    support, cols_flat = pl.pallas_call(
        _mm_body,
        grid=(5,),
        in_specs=[
            pl.BlockSpec((N // 5, D), lambda i: (i, 0)),
            pl.BlockSpec((D, D), lambda i: (0, 0)),
            pl.BlockSpec((1, D), lambda i: (0, 0)),
            pl.BlockSpec((2, E), lambda i: (0, 0)),
        ],
        out_specs=[
            pl.BlockSpec((N // 5, D), lambda i: (i, 0)),
            pl.BlockSpec((E,), lambda i: (0,)),
        ],
        out_shape=[
            jax.ShapeDtypeStruct((N, D), jnp.float32),
            jax.ShapeDtypeStruct((E,), jnp.int32),
        ],
    )(features, W, b2, laplacian_indices)

    partials = _sc_spmm(support, laplacian_indices, cols_flat,
                        laplacian_values)

    out = pl.pallas_call(
        _add_body,
        grid=(10,),
        in_specs=[pl.BlockSpec((NC, N // 10, D), lambda i: (0, i, 0))],
        out_specs=pl.BlockSpec((N // 10, D), lambda i: (i, 0)),
        out_shape=jax.ShapeDtypeStruct((N, D), jnp.float32),
    )(partials)
    return out


# speedup vs baseline: 1.1610x; 1.0054x over previous
import functools

import jax
import jax.numpy as jnp
from jax import lax
from jax.experimental import pallas as pl
from jax.experimental.pallas import tpu as pltpu
from jax.experimental.pallas import tpu_sc as plsc

N = 10000
E = 320000
D = 128

NC = 2
NS = 16
NW = NC * NS
C = 128
NCHUNK = 78
NPAIR = NCHUNK // 2
XTRA_WORKERS = E // C - NW * NCHUNK
WB = 80
RPT = 640
RPT_LAST_CHUNKS = (N - (NS - 1) * RPT) // WB
RPT_CHUNKS = RPT // WB


def _mm_body(f_ref, w_ref, b_ref, lap_ref, o_ref, cols_ref):
    o_ref[...] = (
        jax.lax.dot_general(
            f_ref[...], w_ref[...], (((1,), (1,)), ((), ())),
            preferred_element_type=jnp.float32)
        + b_ref[...]
    )
    @pl.when(pl.program_id(0) == 0)
    def _():
        cols_ref[...] = lap_ref[1, :]


def _add_body(p_ref, o_ref):
    o_ref[...] = p_ref[0] + p_ref[1]


def _bcast_lane(v, lane):
    return lax.gather(
        v, jnp.full((16, 1), lane, jnp.int32),
        lax.GatherDimensionNumbers(
            offset_dims=(), collapsed_slice_dims=(0,), start_index_map=(0,)),
        (1,), mode=lax.GatherScatterMode.PROMISE_IN_BOUNDS)


def _sc_spmm_body(support_hbm, lap_hbm, cols_hbm, vals_hbm, out_hbm,
                  cbuf, rbufa, rbufb, lbufm, vbufa, vbufb, bufa, bufb,
                  acc, gsema, gsemb, esema, esemb, ssem):
    cid = lax.axis_index("c")
    sid = lax.axis_index("s")
    wid = cid * NS + sid
    base_w = (NCHUNK * wid + jnp.minimum(wid, XTRA_WORKERS)) * C
    zero16 = jnp.zeros((16,), jnp.float32)

    pltpu.async_copy(cols_hbm.at[pl.ds(base_w, NCHUNK * C)], cbuf, esema)

    def zrow(g, carry):
        for j in range(D // 16):
            bufa[g, pl.ds(j * 16, 16)] = zero16
        return carry
    lax.fori_loop(0, C, zrow, 0)

    row0 = sid * RPT
    nwb = jnp.where(sid == NS - 1, RPT_LAST_CHUNKS, RPT_CHUNKS)

    def zacc(k, carry):
        pltpu.sync_copy(bufa.at[pl.ds(0, WB)], acc.at[pl.ds(row0 + k * WB, WB)])
        return carry
    lax.fori_loop(0, nwb, zacc, 0)

    plsc.subcore_barrier()
    pltpu.make_async_copy(
        cols_hbm.at[pl.ds(base_w, NCHUNK * C)], cbuf, esema).wait()

    def start_all(ci, buf, rbuf, vbuf, gsem, esem):
        pltpu.async_copy(
            support_hbm.at[cbuf.at[pl.ds(ci * C, C)]], buf, gsem)
        pltpu.async_copy(lap_hbm.at[0, pl.ds(base_w + ci * C, C)], rbuf, esem)
        pltpu.async_copy(vals_hbm.at[pl.ds(base_w + ci * C, C)], vbuf, esem)

    def wait_all(ci, buf, rbuf, vbuf, gsem, esem):
        pltpu.make_async_copy(
            support_hbm.at[cbuf.at[pl.ds(ci * C, C)]], buf, gsem).wait()
        pltpu.make_async_copy(
            lap_hbm.at[0, pl.ds(base_w + ci * C, C)], rbuf, esem).wait()
        pltpu.make_async_copy(
            vals_hbm.at[pl.ds(base_w + ci * C, C)], vbuf, esem).wait()

    def scale(bufg, vbuf, ngroup):
        def body(g, carry):
            vv = vbuf[pl.ds(g * 16, 16)]
            for i2 in range(16):
                r = g * 16 + i2
                s = _bcast_lane(vv, i2)
                for j in range(D // 16):
                    bufg[r, pl.ds(j * 16, 16)] = bufg[r, pl.ds(j * 16, 16)] * s
            return carry
        lax.fori_loop(0, ngroup, body, 0)

    def scatter(buf, rbuf):
        pltpu.async_copy(buf, acc.at[rbuf], ssem, add=True).wait()

    start_all(0, bufa, rbufa, vbufa, gsema, esema)

    def pair(k, carry):
        ci = 2 * k
        wait_all(ci, bufa, rbufa, vbufa, gsema, esema)
        start_all(ci + 1, bufb, rbufb, vbufb, gsemb, esemb)
        scale(bufa, vbufa, C // 16)
        scatter(bufa, rbufa)
        wait_all(ci + 1, bufb, rbufb, vbufb, gsemb, esemb)

        @pl.when(k < NPAIR - 1)
        def _():
            start_all(ci + 2, bufa, rbufa, vbufa, gsema, esema)
        scale(bufb, vbufb, C // 16)
        scatter(bufb, rbufb)
        return carry
    lax.fori_loop(0, NPAIR, pair, 0)

    @pl.when(wid < XTRA_WORKERS)
    def _():
        pltpu.sync_copy(lap_hbm.at[:, pl.ds(base_w + NCHUNK * C, C)], lbufm)
        pltpu.sync_copy(vals_hbm.at[pl.ds(base_w + NCHUNK * C, C)], vbufa)
        pltpu.async_copy(support_hbm.at[lbufm.at[1]], bufa, gsema).wait()
        scale(bufa, vbufa, C // 16)
        pltpu.async_copy(bufa, acc.at[lbufm.at[0]], ssem, add=True).wait()

    plsc.subcore_barrier()

    def wb(k, carry):
        sl = pl.ds(row0 + k * WB, WB)
        pltpu.sync_copy(acc.at[sl], out_hbm.at[cid, sl])
        return carry
    lax.fori_loop(0, nwb, wb, 0)


_sc_spmm = functools.partial(
    pl.kernel,
    out_type=jax.ShapeDtypeStruct((NC, N, D), jnp.float32),
    mesh=plsc.VectorSubcoreMesh(
        core_axis_name="c", subcore_axis_name="s",
        num_cores=NC, num_subcores=NS),
    scratch_types=[
        pltpu.VMEM((NCHUNK * C,), jnp.int32),
        pltpu.VMEM((C,), jnp.int32),
        pltpu.VMEM((C,), jnp.int32),
        pltpu.VMEM((2, C), jnp.int32),
        pltpu.VMEM((C,), jnp.float32),
        pltpu.VMEM((C,), jnp.float32),
        pltpu.VMEM((C, D), jnp.float32),
        pltpu.VMEM((C, D), jnp.float32),
        pltpu.VMEM_SHARED((N, D), jnp.float32),
        pltpu.SemaphoreType.DMA,
        pltpu.SemaphoreType.DMA,
        pltpu.SemaphoreType.DMA,
        pltpu.SemaphoreType.DMA,
        pltpu.SemaphoreType.DMA,
    ],
)(_sc_spmm_body)


def kernel(laplacian_indices, laplacian_values, features, W, b):
    b2 = b.reshape(1, D)

    support, cols_flat = pl.pallas_call(
        _mm_body,
        grid=(5,),
        in_specs=[
            pl.BlockSpec((N // 5, D), lambda i: (i, 0)),
            pl.BlockSpec((D, D), lambda i: (0, 0)),
            pl.BlockSpec((1, D), lambda i: (0, 0)),
            pl.BlockSpec((2, E), lambda i: (0, 0)),
        ],
        out_specs=[
            pl.BlockSpec((N // 5, D), lambda i: (i, 0)),
            pl.BlockSpec((E,), lambda i: (0,)),
        ],
        out_shape=[
            jax.ShapeDtypeStruct((N, D), jnp.float32),
            jax.ShapeDtypeStruct((E,), jnp.int32),
        ],
    )(features, W, b2, laplacian_indices)

    partials = _sc_spmm(support, laplacian_indices, cols_flat,
                        laplacian_values)

    out = pl.pallas_call(
        _add_body,
        grid=(10,),
        in_specs=[pl.BlockSpec((NC, N // 10, D), lambda i: (0, i, 0))],
        out_specs=pl.BlockSpec((N // 10, D), lambda i: (i, 0)),
        out_shape=jax.ShapeDtypeStruct((N, D), jnp.float32),
    )(partials)
    return out
